# baseline (speedup 1.0000x reference)
import functools

import jax
import jax.numpy as jnp
from jax import lax
from jax.experimental import pallas as pl
from jax.experimental.pallas import tpu as pltpu
from jax.experimental.pallas import tpu_sc as plsc

N = 10000
E = 320000
D = 128
DX = 144

NC = 2
NS = 16
NW = NC * NS
NP = 10240
EPW = E // NW
EPW2 = 10240
PADW = EPW2 - EPW
B = 80
NCHUNK = EPW2 // B
CPS = 32
NST = NCHUNK // CPS
PAIRS = CPS // 2
ROWS_PT = NP // NS


def _pre_body(f_ref, w_ref, al_ref, ar_ref, hx_ref, er_ref):
    h = jnp.dot(f_ref[...], w_ref[...], preferred_element_type=jnp.float32)
    el = jnp.sum(h * al_ref[...], axis=1, keepdims=True)
    col = lax.broadcasted_iota(jnp.int32, (h.shape[0], DX - D), 1)
    extra = jnp.where(col == 0, 1.0, jnp.where(col == 1, el, 0.0))
    hx_ref[...] = jnp.concatenate([h, extra.astype(jnp.float32)], axis=1)
    er_ref[...] = jnp.sum(h * ar_ref[...], axis=1, keepdims=True)


def _pre(feats, W, attn_l, attn_r):
    blk = 1000
    return pl.pallas_call(
        _pre_body,
        grid=(N // blk,),
        in_specs=[
            pl.BlockSpec((blk, D), lambda i: (i, 0)),
            pl.BlockSpec((D, D), lambda i: (0, 0)),
            pl.BlockSpec((1, D), lambda i: (0, 0)),
            pl.BlockSpec((1, D), lambda i: (0, 0)),
        ],
        out_specs=[
            pl.BlockSpec((blk, DX), lambda i: (i, 0)),
            pl.BlockSpec((blk, 1), lambda i: (i, 0)),
        ],
        out_shape=[
            jax.ShapeDtypeStruct((N, DX), jnp.float32),
            jax.ShapeDtypeStruct((N, 1), jnp.float32),
        ],
    )(feats, W, attn_l.reshape(1, D), attn_r.reshape(1, D))


_MESH = plsc.VectorSubcoreMesh(core_axis_name="c", subcore_axis_name="s")


@functools.partial(
    pl.kernel,
    out_type=jax.ShapeDtypeStruct((NC, NP, DX), jnp.float32),
    mesh=_MESH,
    compiler_params=pltpu.CompilerParams(use_tc_tiling_on_sc=False,
                                         needs_layout_passes=False),
    scratch_types=[
        pltpu.VMEM((NP,), jnp.float32),
        pltpu.VMEM((CPS, B), jnp.int32),
        pltpu.VMEM((CPS, B), jnp.int32),
        pltpu.VMEM((B,), jnp.float32),
        pltpu.VMEM((B, DX), jnp.float32),
        pltpu.VMEM((B, DX), jnp.float32),
        pltpu.VMEM_SHARED((NP, DX), jnp.float32),
        pltpu.SemaphoreType.DMA,
        pltpu.SemaphoreType.DMA,
        pltpu.SemaphoreType.DMA,
        pltpu.SemaphoreType.DMA,
    ],
)
def _sc_edge(hx_hbm, src_hbm, dst_hbm, er_hbm, acc_hbm,
             er_v, si_v, di_v, ex_v, rows0, rows1, acc_sh,
             g0, g1, c0, c1):
    c = lax.axis_index("c")
    s = lax.axis_index("s")
    w = c * NS + s

    pltpu.sync_copy(er_hbm, er_v)

    zv = jnp.zeros((16,), jnp.float32)
    def _zero_row(i, _):
        for k in range(DX // 16):
            rows0[i, pl.ds(k * 16, 16)] = zv
        return 0
    lax.fori_loop(0, B, _zero_row, 0)
    r0 = s * ROWS_PT
    for p in range(ROWS_PT // B):
        pltpu.sync_copy(rows0, acc_sh.at[pl.ds(r0 + p * B, B)])
    plsc.subcore_barrier()

    def gather(t, buf, sem):
        pltpu.async_copy(hx_hbm.at[si_v.at[t]], buf, sem)

    def wait_gather(t, buf, sem):
        pltpu.make_async_copy(hx_hbm.at[si_v.at[t]], buf, sem).wait()

    def scatter(t, buf, sem):
        pltpu.async_copy(buf, acc_sh.at[di_v.at[t]], sem, add=True)

    def wait_scatter(t, buf, sem):
        pltpu.make_async_copy(buf, acc_sh.at[di_v.at[t]], sem).wait()

    def process(t, buf):
        for g in range(B // 16):
            sl = pl.ds(g * 16, 16)
            rid = lax.broadcasted_iota(jnp.int32, (16,), 0) + g * 16
            elv = plsc.load_gather(buf, [rid, jnp.full((16,), D + 1, jnp.int32)])
            erv = plsc.load_gather(er_v, [di_v[t, sl]])
            z = elv + erv
            z = jnp.where(z >= 0, z, 0.2 * z)
            ex_v[sl] = jnp.exp(z)
        for g in range(B // 16):
            ex16 = ex_v[pl.ds(g * 16, 16)]
            for j in range(16):
                i = g * 16 + j
                bex = ex16.at[jnp.full((16,), j, jnp.int32)].get(
                    mode='promise_in_bounds')
                for k in range(D // 16):
                    sl = pl.ds(k * 16, 16)
                    buf[i, sl] = buf[i, sl] * bex
                buf[i, pl.ds(D, 16)] = bex

    def _stage(ts, _):
        pltpu.sync_copy(src_hbm.at[w, pl.ds(ts * CPS, CPS)], si_v)
        pltpu.sync_copy(dst_hbm.at[w, pl.ds(ts * CPS, CPS)], di_v)
        gather(0, rows0, g0)
        wait_gather(0, rows0, g0)
        process(0, rows0)
        scatter(0, rows0, c0)
        gather(1, rows1, g1)
        wait_gather(1, rows1, g1)
        process(1, rows1)
        scatter(1, rows1, c1)
        wait_scatter(0, rows0, c0)
        gather(2, rows0, g0)

        def _pair(t2, _):
            t0 = 2 * t2
            t1 = t0 + 1
            wait_gather(t0, rows0, g0)
            process(t0, rows0)
            scatter(t0, rows0, c0)
            wait_scatter(t1 - 2, rows1, c1)
            gather(t1, rows1, g1)
            wait_gather(t1, rows1, g1)
            process(t1, rows1)
            scatter(t1, rows1, c1)
            wait_scatter(t0, rows0, c0)
            @pl.when(t2 != PAIRS - 1)
            def _():
                gather(t0 + 2, rows0, g0)
            return 0

        lax.fori_loop(1, PAIRS, _pair, 0)
        wait_scatter(CPS - 1, rows1, c1)
        return 0

    lax.fori_loop(0, NST, _stage, 0)
    plsc.subcore_barrier()

    pltpu.sync_copy(acc_sh.at[pl.ds(r0, ROWS_PT)],
                    acc_hbm.at[c, pl.ds(r0, ROWS_PT)])


def _post_body(acc_ref, b_ref, out_ref):
    num = acc_ref[0, :, :D] + acc_ref[1, :, :D]
    sv = acc_ref[0, :, D:D + 1] + acc_ref[1, :, D:D + 1]
    out_ref[...] = num / (sv + 1e-9) + b_ref[...]


def _post(acc, bias):
    blk = 1000
    return pl.pallas_call(
        _post_body,
        grid=(N // blk,),
        in_specs=[
            pl.BlockSpec((NC, blk, DX), lambda i: (0, i, 0)),
            pl.BlockSpec((1, D), lambda i: (0, 0)),
        ],
        out_specs=pl.BlockSpec((blk, D), lambda i: (i, 0)),
        out_shape=jax.ShapeDtypeStruct((N, D), jnp.float32),
    )(acc, bias.reshape(1, D))


def kernel(feats, edge_index, W, attn_l, attn_r, bias):
    src = edge_index[0].reshape(NW, EPW)
    dst = edge_index[1].reshape(NW, EPW)
    pad_src = jnp.zeros((NW, PADW), jnp.int32)
    pad_dst = jnp.broadcast_to(
        N + jnp.arange(PADW, dtype=jnp.int32)[None, :], (NW, PADW))
    src = jnp.concatenate([src, pad_src], axis=1).reshape(NW, NCHUNK, B)
    dst = jnp.concatenate([dst, pad_dst], axis=1).reshape(NW, NCHUNK, B)
    hx, er = _pre(feats, W, attn_l, attn_r)
    er_pad = jnp.pad(er.reshape(N), (0, NP - N))
    acc = _sc_edge(hx, src, dst, er_pad)
    out = _post(acc, bias)
    return out.reshape(N, 1, D)


# speedup vs baseline: 1.8083x; 1.8083x over previous
import functools

import numpy as np
import jax
import jax.numpy as jnp
from jax import lax
from jax.experimental import pallas as pl
from jax.experimental.pallas import tpu as pltpu
from jax.experimental.pallas import tpu_sc as plsc

N = 10000
E = 320000
D = 128
DX = 144

NC = 2
NS = 16
NW = NC * NS
EPW = E // NW
B = 80
NCHUNK = EPW // B
CPS = 25
NST = NCHUNK // CPS
ROWS_PT = N // NS

_PERM = np.zeros(D, np.int32)
for _kk in range(D // 32):
    for _j in range(16):
        _PERM[32 * _kk + _j] = 32 * _kk + 2 * _j
        _PERM[32 * _kk + 16 + _j] = 32 * _kk + 2 * _j + 1
_IPERM = np.argsort(_PERM).astype(np.int32)


def _pre_body(f_ref, w_ref, al_ref, ar_ref, h16_ref, el_ref, er_ref):
    h = jnp.dot(f_ref[...], w_ref[...], preferred_element_type=jnp.float32)
    h16_ref[...] = h.astype(jnp.bfloat16)
    el_ref[...] = jnp.sum(h * al_ref[...], axis=1, keepdims=True)
    er_ref[...] = jnp.sum(h * ar_ref[...], axis=1, keepdims=True)


def _pre(feats, W, attn_l, attn_r):
    blk = 1000
    return pl.pallas_call(
        _pre_body,
        grid=(N // blk,),
        in_specs=[
            pl.BlockSpec((blk, D), lambda i: (i, 0)),
            pl.BlockSpec((D, D), lambda i: (0, 0)),
            pl.BlockSpec((1, D), lambda i: (0, 0)),
            pl.BlockSpec((1, D), lambda i: (0, 0)),
        ],
        out_specs=[
            pl.BlockSpec((blk, D), lambda i: (i, 0)),
            pl.BlockSpec((blk, 1), lambda i: (i, 0)),
            pl.BlockSpec((blk, 1), lambda i: (i, 0)),
        ],
        out_shape=[
            jax.ShapeDtypeStruct((N, D), jnp.bfloat16),
            jax.ShapeDtypeStruct((N, 1), jnp.float32),
            jax.ShapeDtypeStruct((N, 1), jnp.float32),
        ],
    )(feats, W, attn_l.reshape(1, D), attn_r.reshape(1, D))


_MESH = plsc.VectorSubcoreMesh(core_axis_name="c", subcore_axis_name="s")


@functools.partial(
    pl.kernel,
    out_type=jax.ShapeDtypeStruct((NC, N, DX), jnp.float32),
    mesh=_MESH,
    compiler_params=pltpu.CompilerParams(use_tc_tiling_on_sc=False,
                                         needs_layout_passes=False),
    scratch_types=[
        pltpu.VMEM((N,), jnp.float32),
        pltpu.VMEM((N,), jnp.float32),
        pltpu.VMEM((CPS, B), jnp.int32),
        pltpu.VMEM((CPS, B), jnp.int32),
        pltpu.VMEM((B,), jnp.float32),
        pltpu.VMEM((B, D), jnp.bfloat16),
        pltpu.VMEM((B, DX), jnp.float32),
        pltpu.VMEM_SHARED((N, DX), jnp.float32),
        pltpu.SemaphoreType.DMA,
    ],
)
def _sc_edge(h16_hbm, src_hbm, dst_hbm, el_hbm, er_hbm, acc_hbm,
             el_v, er_v, si_v, di_v, ex_v, gbuf, sbuf, acc_sh, sem):
    c = lax.axis_index("c")
    s = lax.axis_index("s")
    w = c * NS + s

    pltpu.sync_copy(el_hbm, el_v)
    pltpu.sync_copy(er_hbm, er_v)

    zv = jnp.zeros((16,), jnp.float32)
    def _zero_row(i, _):
        for k in range(DX // 16):
            sbuf[i, pl.ds(k * 16, 16)] = zv
        return 0
    lax.fori_loop(0, B, _zero_row, 0)
    r0 = s * ROWS_PT
    for p in range(ROWS_PT // B):
        pltpu.sync_copy(sbuf, acc_sh.at[pl.ds(r0 + p * B, B)])
    pltpu.sync_copy(sbuf.at[pl.ds(0, ROWS_PT % B)],
                    acc_sh.at[pl.ds(r0 + (ROWS_PT // B) * B, ROWS_PT % B)])
    plsc.subcore_barrier()

    def _chunk(t, _):
        cp = pltpu.async_copy(h16_hbm.at[si_v.at[t]], gbuf, sem)
        for g in range(B // 16):
            sl = pl.ds(g * 16, 16)
            isrc = si_v[t, sl]
            idst = di_v[t, sl]
            z = plsc.load_gather(el_v, [isrc]) + plsc.load_gather(er_v, [idst])
            z = jnp.where(z >= 0, z, 0.2 * z)
            ex_v[sl] = jnp.exp(z)
        cp.wait()
        for g in range(B // 16):
            ex16 = ex_v[pl.ds(g * 16, 16)]
            for j in range(16):
                i = g * 16 + j
                bex = ex16.at[jnp.full((16,), j, jnp.int32)].get(
                    mode='promise_in_bounds')
                for kk in range(D // 32):
                    v32 = gbuf[i, pl.ds(kk * 32, 32)]
                    a, b = plsc.unpack(v32, format=plsc.PackFormat.INTERLEAVED)
                    sbuf[i, pl.ds(kk * 32, 16)] = a * bex
                    sbuf[i, pl.ds(kk * 32 + 16, 16)] = b * bex
                sbuf[i, pl.ds(D, 16)] = bex
        pltpu.sync_copy(sbuf, acc_sh.at[di_v.at[t]], add=True)
        return 0

    def _stage(ts, _):
        pltpu.sync_copy(src_hbm.at[w, pl.ds(ts * CPS, CPS)], si_v)
        pltpu.sync_copy(dst_hbm.at[w, pl.ds(ts * CPS, CPS)], di_v)
        lax.fori_loop(0, CPS, _chunk, 0)
        return 0

    lax.fori_loop(0, NST, _stage, 0)
    plsc.subcore_barrier()

    pltpu.sync_copy(acc_sh.at[pl.ds(r0, ROWS_PT)],
                    acc_hbm.at[c, pl.ds(r0, ROWS_PT)])


def _post_body(acc_ref, b_ref, out_ref):
    num = acc_ref[0, :, :D] + acc_ref[1, :, :D]
    sv = acc_ref[0, :, D:D + 1] + acc_ref[1, :, D:D + 1]
    out_ref[...] = num / (sv + 1e-9) + b_ref[...]


def _post(acc, bias_p):
    blk = 1000
    return pl.pallas_call(
        _post_body,
        grid=(N // blk,),
        in_specs=[
            pl.BlockSpec((NC, blk, DX), lambda i: (0, i, 0)),
            pl.BlockSpec((1, D), lambda i: (0, 0)),
        ],
        out_specs=pl.BlockSpec((blk, D), lambda i: (i, 0)),
        out_shape=jax.ShapeDtypeStruct((N, D), jnp.float32),
    )(acc, bias_p.reshape(1, D))


def kernel(feats, edge_index, W, attn_l, attn_r, bias):
    src = edge_index[0].reshape(NW, NCHUNK, B)
    dst = edge_index[1].reshape(NW, NCHUNK, B)
    h16, el, er = _pre(feats, W, attn_l, attn_r)
    acc = _sc_edge(h16, src, dst, el.reshape(N), er.reshape(N))
    out_p = _post(acc, bias[jnp.asarray(_PERM)])
    out = jnp.take(out_p, jnp.asarray(_IPERM), axis=1)
    return out.reshape(N, 1, D)


# speedup vs baseline: 2.1434x; 1.1853x over previous
import functools

import numpy as np
import jax
import jax.numpy as jnp
from jax import lax
from jax.experimental import pallas as pl
from jax.experimental.pallas import tpu as pltpu
from jax.experimental.pallas import tpu_sc as plsc

N = 10000
E = 320000
D = 128
DX = 144

NC = 2
NS = 16
NW = NC * NS
EPW = E // NW
B = 80
NCHUNK = EPW // B
CPS = 25
NST = NCHUNK // CPS
ROWS_PT = N // NS

_PERM = np.zeros(D, np.int32)
for _kk in range(D // 32):
    for _j in range(16):
        _PERM[32 * _kk + _j] = 32 * _kk + 2 * _j
        _PERM[32 * _kk + 16 + _j] = 32 * _kk + 2 * _j + 1
_IPERM = np.argsort(_PERM).astype(np.int32)


def _pre_body(f_ref, w_ref, al_ref, ar_ref, h16_ref, el_ref, er_ref):
    h = jnp.dot(f_ref[...], w_ref[...], preferred_element_type=jnp.float32)
    h16_ref[...] = h.astype(jnp.bfloat16)
    el_ref[...] = jnp.sum(h * al_ref[...], axis=1, keepdims=True)
    er_ref[...] = jnp.sum(h * ar_ref[...], axis=1, keepdims=True)


def _pre(feats, W, attn_l, attn_r):
    blk = 1000
    return pl.pallas_call(
        _pre_body,
        grid=(N // blk,),
        in_specs=[
            pl.BlockSpec((blk, D), lambda i: (i, 0)),
            pl.BlockSpec((D, D), lambda i: (0, 0)),
            pl.BlockSpec((1, D), lambda i: (0, 0)),
            pl.BlockSpec((1, D), lambda i: (0, 0)),
        ],
        out_specs=[
            pl.BlockSpec((blk, D), lambda i: (i, 0)),
            pl.BlockSpec((blk, 1), lambda i: (i, 0)),
            pl.BlockSpec((blk, 1), lambda i: (i, 0)),
        ],
        out_shape=[
            jax.ShapeDtypeStruct((N, D), jnp.bfloat16),
            jax.ShapeDtypeStruct((N, 1), jnp.float32),
            jax.ShapeDtypeStruct((N, 1), jnp.float32),
        ],
    )(feats, W, attn_l.reshape(1, D), attn_r.reshape(1, D))


_MESH = plsc.VectorSubcoreMesh(core_axis_name="c", subcore_axis_name="s")


@functools.partial(
    pl.kernel,
    out_type=jax.ShapeDtypeStruct((NC, N, DX), jnp.float32),
    mesh=_MESH,
    compiler_params=pltpu.CompilerParams(use_tc_tiling_on_sc=False,
                                         needs_layout_passes=False),
    scratch_types=[
        pltpu.VMEM((N,), jnp.float32),
        pltpu.VMEM((N,), jnp.float32),
        pltpu.VMEM((CPS, B), jnp.int32),
        pltpu.VMEM((CPS, B), jnp.int32),
        pltpu.VMEM((B,), jnp.float32),
        pltpu.VMEM((B, D), jnp.bfloat16),
        pltpu.VMEM((B, DX), jnp.float32),
        pltpu.VMEM_SHARED((N, DX), jnp.float32),
        pltpu.SemaphoreType.DMA,
        pltpu.SemaphoreType.DMA,
    ],
)
def _sc_edge(h16_hbm, src_hbm, dst_hbm, el_hbm, er_hbm, acc_hbm,
             el_v, er_v, si_v, di_v, ex_v, gbuf, sbuf, acc_sh, sem, csem):
    c = lax.axis_index("c")
    s = lax.axis_index("s")
    w = c * NS + s

    pltpu.sync_copy(el_hbm, el_v)
    pltpu.sync_copy(er_hbm, er_v)

    zv = jnp.zeros((16,), jnp.float32)
    def _zero_row(i, _):
        for k in range(DX // 16):
            sbuf[i, pl.ds(k * 16, 16)] = zv
        return 0
    lax.fori_loop(0, B, _zero_row, 0)
    r0 = s * ROWS_PT
    for p in range(ROWS_PT // B):
        pltpu.sync_copy(sbuf, acc_sh.at[pl.ds(r0 + p * B, B)])
    pltpu.sync_copy(sbuf.at[pl.ds(0, ROWS_PT % B)],
                    acc_sh.at[pl.ds(r0 + (ROWS_PT // B) * B, ROWS_PT % B)])
    plsc.subcore_barrier()

    def _chunk(t, _):
        cp = pltpu.async_copy(h16_hbm.at[si_v.at[t]], gbuf, sem)
        for g in range(B // 16):
            sl = pl.ds(g * 16, 16)
            isrc = si_v[t, sl]
            idst = di_v[t, sl]
            z = plsc.load_gather(el_v, [isrc]) + plsc.load_gather(er_v, [idst])
            z = jnp.where(z >= 0, z, 0.2 * z)
            ex_v[sl] = jnp.exp(z)
        cp.wait()
        @pl.when(t > 0)
        def _():
            pltpu.make_async_copy(sbuf, acc_sh.at[di_v.at[t - 1]], csem).wait()
        for g in range(B // 16):
            ex16 = ex_v[pl.ds(g * 16, 16)]
            for j in range(16):
                i = g * 16 + j
                bex = ex16.at[jnp.full((16,), j, jnp.int32)].get(
                    mode='promise_in_bounds')
                for kk in range(D // 32):
                    v32 = gbuf[i, pl.ds(kk * 32, 32)]
                    a, b = plsc.unpack(v32, format=plsc.PackFormat.INTERLEAVED)
                    sbuf[i, pl.ds(kk * 32, 16)] = a * bex
                    sbuf[i, pl.ds(kk * 32 + 16, 16)] = b * bex
                sbuf[i, pl.ds(D, 16)] = bex
        pltpu.async_copy(sbuf, acc_sh.at[di_v.at[t]], csem, add=True)
        return 0

    def _stage(ts, _):
        pltpu.sync_copy(src_hbm.at[w, pl.ds(ts * CPS, CPS)], si_v)
        pltpu.sync_copy(dst_hbm.at[w, pl.ds(ts * CPS, CPS)], di_v)
        lax.fori_loop(0, CPS, _chunk, 0)
        pltpu.make_async_copy(sbuf, acc_sh.at[di_v.at[CPS - 1]], csem).wait()
        return 0

    lax.fori_loop(0, NST, _stage, 0)
    plsc.subcore_barrier()

    pltpu.sync_copy(acc_sh.at[pl.ds(r0, ROWS_PT)],
                    acc_hbm.at[c, pl.ds(r0, ROWS_PT)])


def _post_body(acc_ref, b_ref, out_ref):
    num = acc_ref[0, :, :D] + acc_ref[1, :, :D]
    sv = acc_ref[0, :, D:D + 1] + acc_ref[1, :, D:D + 1]
    out_ref[...] = num / (sv + 1e-9) + b_ref[...]


def _post(acc, bias_p):
    blk = 1000
    return pl.pallas_call(
        _post_body,
        grid=(N // blk,),
        in_specs=[
            pl.BlockSpec((NC, blk, DX), lambda i: (0, i, 0)),
            pl.BlockSpec((1, D), lambda i: (0, 0)),
        ],
        out_specs=pl.BlockSpec((blk, D), lambda i: (i, 0)),
        out_shape=jax.ShapeDtypeStruct((N, D), jnp.float32),
    )(acc, bias_p.reshape(1, D))


def kernel(feats, edge_index, W, attn_l, attn_r, bias):
    src = edge_index[0].reshape(NW, NCHUNK, B)
    dst = edge_index[1].reshape(NW, NCHUNK, B)
    h16, el, er = _pre(feats, W, attn_l, attn_r)
    acc = _sc_edge(h16, src, dst, el.reshape(N), er.reshape(N))
    out_p = _post(acc, bias[jnp.asarray(_PERM)])
    out = jnp.take(out_p, jnp.asarray(_IPERM), axis=1)
    return out.reshape(N, 1, D)
